# baseline (speedup 1.0000x reference)
import jax
import jax.numpy as jnp
from jax import lax
from jax.experimental import pallas as pl
from jax.experimental.pallas import tpu as pltpu
from jax.experimental.pallas import tpu_sc as plsc

VOCAB = 1000000
EMBED_DIM = 64
B_TOTAL = 4096 * 200

_info = plsc.get_sparse_core_info()
NC = _info.num_cores
NS = _info.num_subcores
NW = NC * NS

PER_W = B_TOTAL // NW
CHUNK = 128
NCHUNK = PER_W // CHUNK

assert PER_W * NW == B_TOTAL
assert NCHUNK * CHUNK == PER_W


def _embed_body(table_hbm, idx_hbm, out_hbm, idx_v, rows_v, sem):
    wid = lax.axis_index("s") * NC + lax.axis_index("c")
    base = wid * PER_W
    pltpu.sync_copy(idx_hbm.at[pl.ds(base, PER_W)], idx_v)

    @pl.loop(0, NCHUNK)
    def _(i):
        off = i * CHUNK
        pltpu.async_copy(
            table_hbm.at[idx_v.at[pl.ds(off, CHUNK)]], rows_v, sem
        ).wait()
        pltpu.sync_copy(rows_v, out_hbm.at[pl.ds(base + off, CHUNK)])


_mesh = plsc.VectorSubcoreMesh(core_axis_name="c", subcore_axis_name="s")

_embed = pl.kernel(
    _embed_body,
    out_type=jax.ShapeDtypeStruct((B_TOTAL, EMBED_DIM), jnp.float32),
    mesh=_mesh,
    scratch_types=[
        pltpu.VMEM((PER_W,), jnp.int32),
        pltpu.VMEM((CHUNK, EMBED_DIM), jnp.float32),
        pltpu.SemaphoreType.DMA,
    ],
    compiler_params=pltpu.CompilerParams(use_tc_tiling_on_sc=False),
)


@jax.jit
def kernel(input_sentence, word_embedding):
    S, T = input_sentence.shape
    idx = input_sentence.reshape(-1).astype(jnp.int32)
    out = _embed(word_embedding, idx)
    return out.reshape(S, T, EMBED_DIM)


# speedup vs baseline: 1.1120x; 1.1120x over previous
import jax
import jax.numpy as jnp
from jax import lax
from jax.experimental import pallas as pl
from jax.experimental.pallas import tpu as pltpu
from jax.experimental.pallas import tpu_sc as plsc

VOCAB = 1000000
EMBED_DIM = 64
B_TOTAL = 4096 * 200

_info = plsc.get_sparse_core_info()
NC = _info.num_cores
NS = _info.num_subcores
NW = NC * NS

PER_W = B_TOTAL // NW
CHUNK = 128
K = 5
SUPER = K * CHUNK
G = PER_W // SUPER

assert PER_W * NW == B_TOTAL
assert G * SUPER == PER_W and G % 2 == 0


def _embed_body(table_hbm, idx_hbm, out_hbm,
                idx_v, rows0, rows1, gsem0, gsem1, ssem0, ssem1):
    wid = lax.axis_index("s") * NC + lax.axis_index("c")
    base = wid * PER_W
    pltpu.sync_copy(idx_hbm.at[pl.ds(base, PER_W)], idx_v)

    rows = (rows0, rows1)
    gsem = (gsem0, gsem1)
    ssem = (ssem0, ssem1)

    def fire_gathers(g, b):
        off = g * SUPER
        for k in range(K):
            pltpu.async_copy(
                table_hbm.at[idx_v.at[pl.ds(off + k * CHUNK, CHUNK)]],
                rows[b].at[pl.ds(k * CHUNK, CHUNK)],
                gsem[b],
            )

    def drain_gathers(b):
        pltpu.make_async_copy(
            table_hbm.at[pl.ds(0, SUPER)], rows[b], gsem[b]
        ).wait()

    def fire_store(g, b):
        pltpu.async_copy(rows[b], out_hbm.at[pl.ds(base + g * SUPER, SUPER)],
                         ssem[b])

    def drain_store(b):
        pltpu.make_async_copy(
            rows[b], out_hbm.at[pl.ds(base, SUPER)], ssem[b]
        ).wait()

    fire_gathers(0, 0)

    @pl.loop(0, G, step=2)
    def _(go):
        for b in (0, 1):
            g = go + b
            nb = 1 - b

            @pl.when(g + 1 < G)
            def _():
                @pl.when(g >= 1)
                def _():
                    drain_store(nb)
                fire_gathers(g + 1, nb)

            drain_gathers(b)
            fire_store(g, b)

    drain_store(0)
    drain_store(1)


_mesh = plsc.VectorSubcoreMesh(core_axis_name="c", subcore_axis_name="s")

_embed = pl.kernel(
    _embed_body,
    out_type=jax.ShapeDtypeStruct((B_TOTAL, EMBED_DIM), jnp.float32),
    mesh=_mesh,
    scratch_types=[
        pltpu.VMEM((PER_W,), jnp.int32),
        pltpu.VMEM((SUPER, EMBED_DIM), jnp.float32),
        pltpu.VMEM((SUPER, EMBED_DIM), jnp.float32),
        pltpu.SemaphoreType.DMA,
        pltpu.SemaphoreType.DMA,
        pltpu.SemaphoreType.DMA,
        pltpu.SemaphoreType.DMA,
    ],
    compiler_params=pltpu.CompilerParams(use_tc_tiling_on_sc=False),
)


@jax.jit
def kernel(input_sentence, word_embedding):
    S, T = input_sentence.shape
    idx = input_sentence.reshape(-1).astype(jnp.int32)
    out = _embed(word_embedding, idx)
    return out.reshape(S, T, EMBED_DIM)


# speedup vs baseline: 1.4804x; 1.3313x over previous
import jax
import jax.numpy as jnp
from jax import lax
from jax.experimental import pallas as pl
from jax.experimental.pallas import tpu as pltpu
from jax.experimental.pallas import tpu_sc as plsc

VOCAB = 1000000
EMBED_DIM = 64
B_TOTAL = 4096 * 200

_info = plsc.get_sparse_core_info()
NC = _info.num_cores
NS = _info.num_subcores
NW = NC * NS

PAD_DIM = 128
PER_W = B_TOTAL // NW
CHUNK = 128
K = 2
SUPER = K * CHUNK
G = PER_W // SUPER

assert PER_W * NW == B_TOTAL
assert G * SUPER == PER_W and G % 2 == 0


def _embed_body(table_hbm, idx_hbm, out_hbm,
                idx_v, rows0, rows1, gsem0, gsem1, ssem0, ssem1):
    wid = lax.axis_index("s") * NC + lax.axis_index("c")
    base = wid * PER_W
    pltpu.sync_copy(idx_hbm.at[pl.ds(base, PER_W)], idx_v)

    rows = (rows0, rows1)
    gsem = (gsem0, gsem1)
    ssem = (ssem0, ssem1)

    def fire_gathers(g, b):
        off = g * SUPER
        for k in range(K):
            pltpu.async_copy(
                table_hbm.at[idx_v.at[pl.ds(off + k * CHUNK, CHUNK)]],
                rows[b].at[pl.ds(k * CHUNK, CHUNK)],
                gsem[b],
            )

    def drain_gathers(b):
        pltpu.make_async_copy(
            table_hbm.at[pl.ds(0, SUPER)], rows[b], gsem[b]
        ).wait()

    def fire_store(g, b):
        pltpu.async_copy(
            rows[b],
            out_hbm.at[pl.ds(base + g * SUPER, SUPER), pl.ds(0, EMBED_DIM)],
            ssem[b],
        )

    def drain_store(b):
        pltpu.make_async_copy(
            rows[b],
            out_hbm.at[pl.ds(base, SUPER), pl.ds(0, EMBED_DIM)],
            ssem[b],
        ).wait()

    fire_gathers(0, 0)

    @pl.loop(0, G, step=2)
    def _(go):
        for b in (0, 1):
            g = go + b
            nb = 1 - b

            @pl.when(g + 1 < G)
            def _():
                @pl.when(g >= 1)
                def _():
                    drain_store(nb)
                fire_gathers(g + 1, nb)

            drain_gathers(b)
            fire_store(g, b)

    drain_store(0)
    drain_store(1)


_mesh = plsc.VectorSubcoreMesh(core_axis_name="c", subcore_axis_name="s")

_embed = pl.kernel(
    _embed_body,
    out_type=jax.ShapeDtypeStruct((B_TOTAL, PAD_DIM), jnp.float32),
    mesh=_mesh,
    scratch_types=[
        pltpu.VMEM((PER_W,), jnp.int32),
        pltpu.VMEM((SUPER, EMBED_DIM), jnp.float32),
        pltpu.VMEM((SUPER, EMBED_DIM), jnp.float32),
        pltpu.SemaphoreType.DMA,
        pltpu.SemaphoreType.DMA,
        pltpu.SemaphoreType.DMA,
        pltpu.SemaphoreType.DMA,
    ],
    compiler_params=pltpu.CompilerParams(use_tc_tiling_on_sc=False),
)


@jax.jit
def kernel(input_sentence, word_embedding):
    S, T = input_sentence.shape
    idx = input_sentence.reshape(-1).astype(jnp.int32)
    out = _embed(word_embedding, idx)
    return out[:, :EMBED_DIM].reshape(S, T, EMBED_DIM)


# speedup vs baseline: 1.4804x; 1.0000x over previous
import jax
import jax.numpy as jnp
from jax import lax
from jax.experimental import pallas as pl
from jax.experimental.pallas import tpu as pltpu
from jax.experimental.pallas import tpu_sc as plsc

VOCAB = 1000000
EMBED_DIM = 64
B_TOTAL = 4096 * 200

_info = plsc.get_sparse_core_info()
NC = _info.num_cores
NS = _info.num_subcores
NW = NC * NS

PAD_DIM = 128
PER_W = B_TOTAL // NW
CHUNK = 256
K = 2
SUPER = K * CHUNK
G = PER_W // SUPER

assert PER_W * NW == B_TOTAL
assert G * SUPER == PER_W and G % 2 == 0


def _embed_body(table_hbm, idx_hbm, out_hbm,
                idx_v, rows0, rows1, gsem0, gsem1, ssem0, ssem1):
    wid = lax.axis_index("s") * NC + lax.axis_index("c")
    base = wid * PER_W
    pltpu.sync_copy(idx_hbm.at[pl.ds(base, PER_W)], idx_v)

    rows = (rows0, rows1)
    gsem = (gsem0, gsem1)
    ssem = (ssem0, ssem1)

    def fire_gathers(g, b):
        off = g * SUPER
        for k in range(K):
            pltpu.async_copy(
                table_hbm.at[idx_v.at[pl.ds(off + k * CHUNK, CHUNK)]],
                rows[b].at[pl.ds(k * CHUNK, CHUNK)],
                gsem[b],
            )

    def drain_gathers(b):
        pltpu.make_async_copy(
            table_hbm.at[pl.ds(0, SUPER)], rows[b], gsem[b]
        ).wait()

    def fire_store(g, b):
        pltpu.async_copy(
            rows[b],
            out_hbm.at[pl.ds(base + g * SUPER, SUPER), pl.ds(0, EMBED_DIM)],
            ssem[b],
        )

    def drain_store(b):
        pltpu.make_async_copy(
            rows[b],
            out_hbm.at[pl.ds(base, SUPER), pl.ds(0, EMBED_DIM)],
            ssem[b],
        ).wait()

    fire_gathers(0, 0)

    @pl.loop(0, G, step=2)
    def _(go):
        for b in (0, 1):
            g = go + b
            nb = 1 - b

            @pl.when(g + 1 < G)
            def _():
                @pl.when(g >= 1)
                def _():
                    drain_store(nb)
                fire_gathers(g + 1, nb)

            drain_gathers(b)
            fire_store(g, b)

    drain_store(0)
    drain_store(1)


_mesh = plsc.VectorSubcoreMesh(core_axis_name="c", subcore_axis_name="s")

_embed = pl.kernel(
    _embed_body,
    out_type=jax.ShapeDtypeStruct((B_TOTAL, PAD_DIM), jnp.float32),
    mesh=_mesh,
    scratch_types=[
        pltpu.VMEM((PER_W,), jnp.int32),
        pltpu.VMEM((SUPER, EMBED_DIM), jnp.float32),
        pltpu.VMEM((SUPER, EMBED_DIM), jnp.float32),
        pltpu.SemaphoreType.DMA,
        pltpu.SemaphoreType.DMA,
        pltpu.SemaphoreType.DMA,
        pltpu.SemaphoreType.DMA,
    ],
    compiler_params=pltpu.CompilerParams(use_tc_tiling_on_sc=False),
)


@jax.jit
def kernel(input_sentence, word_embedding):
    S, T = input_sentence.shape
    idx = input_sentence.reshape(-1).astype(jnp.int32)
    out = _embed(word_embedding, idx)
    return out[:, :EMBED_DIM].reshape(S, T, EMBED_DIM)


# speedup vs baseline: 1.4807x; 1.0002x over previous
import jax
import jax.numpy as jnp
from jax import lax
from jax.experimental import pallas as pl
from jax.experimental.pallas import tpu as pltpu
from jax.experimental.pallas import tpu_sc as plsc

VOCAB = 1000000
EMBED_DIM = 64
B_TOTAL = 4096 * 200

_info = plsc.get_sparse_core_info()
NC = _info.num_cores
NS = _info.num_subcores
NW = NC * NS

PAD_DIM = 128
PER_W = B_TOTAL // NW
CHUNK = 128
K = 2
SUPER = K * CHUNK
G = PER_W // SUPER

assert PER_W * NW == B_TOTAL
assert G * SUPER == PER_W and G % 2 == 0


def _embed_body(table_hbm, idx_hbm, out_hbm,
                idx_v, rows0, rows1, gsem0, gsem1, ssem0, ssem1):
    wid = lax.axis_index("s") * NC + lax.axis_index("c")
    base = wid * PER_W
    pltpu.sync_copy(idx_hbm.at[pl.ds(base, PER_W)], idx_v)

    rows = (rows0, rows1)
    gsem = (gsem0, gsem1)
    ssem = (ssem0, ssem1)

    def fire_gathers(g, b):
        off = g * SUPER
        for k in range(K):
            pltpu.async_copy(
                table_hbm.at[idx_v.at[pl.ds(off + k * CHUNK, CHUNK)]],
                rows[b].at[pl.ds(k * CHUNK, CHUNK)],
                gsem[b],
            )

    def drain_gathers(b):
        pltpu.make_async_copy(
            table_hbm.at[pl.ds(0, SUPER)], rows[b], gsem[b]
        ).wait()

    def fire_store(g, b):
        pltpu.async_copy(
            rows[b],
            out_hbm.at[pl.ds(base + g * SUPER, SUPER), pl.ds(0, EMBED_DIM)],
            ssem[b],
        )

    def drain_store(b):
        pltpu.make_async_copy(
            rows[b],
            out_hbm.at[pl.ds(base, SUPER), pl.ds(0, EMBED_DIM)],
            ssem[b],
        ).wait()

    fire_gathers(0, 0)

    @pl.loop(0, G, step=2)
    def _(go):
        for b in (0, 1):
            g = go + b
            nb = 1 - b

            @pl.when(g + 1 < G)
            def _():
                @pl.when(g >= 1)
                def _():
                    drain_store(nb)
                fire_gathers(g + 1, nb)

            drain_gathers(b)
            fire_store(g, b)

    drain_store(0)
    drain_store(1)


_mesh = plsc.VectorSubcoreMesh(core_axis_name="c", subcore_axis_name="s")

_embed = pl.kernel(
    _embed_body,
    out_type=jax.ShapeDtypeStruct((B_TOTAL, PAD_DIM), jnp.float32),
    mesh=_mesh,
    scratch_types=[
        pltpu.VMEM((PER_W,), jnp.int32),
        pltpu.VMEM((SUPER, EMBED_DIM), jnp.float32),
        pltpu.VMEM((SUPER, EMBED_DIM), jnp.float32),
        pltpu.SemaphoreType.DMA,
        pltpu.SemaphoreType.DMA,
        pltpu.SemaphoreType.DMA,
        pltpu.SemaphoreType.DMA,
    ],
    compiler_params=pltpu.CompilerParams(use_tc_tiling_on_sc=False),
)


@jax.jit
def kernel(input_sentence, word_embedding):
    S, T = input_sentence.shape
    idx = input_sentence.reshape(-1).astype(jnp.int32)
    out = _embed(word_embedding, idx)
    return out[:, :EMBED_DIM].reshape(S, T, EMBED_DIM)


# speedup vs baseline: 1.4816x; 1.0007x over previous
import jax
import jax.numpy as jnp
from jax import lax
from jax.experimental import pallas as pl
from jax.experimental.pallas import tpu as pltpu
from jax.experimental.pallas import tpu_sc as plsc

VOCAB = 1000000
EMBED_DIM = 64
B_TOTAL = 4096 * 200

_info = plsc.get_sparse_core_info()
NC = _info.num_cores
NS = _info.num_subcores
NW = NC * NS

PAD_DIM = 128
PER_W = B_TOTAL // NW
CHUNK = 128
K = 4
SUPER = K * CHUNK
G = PER_W // SUPER

assert PER_W * NW == B_TOTAL
assert G * SUPER == PER_W and G % 2 == 0


def _embed_body(table_hbm, idx_hbm, out_hbm,
                idx_v, rows0, rows1, gsem0, gsem1, ssem0, ssem1):
    wid = lax.axis_index("s") * NC + lax.axis_index("c")
    base = wid * PER_W
    pltpu.sync_copy(idx_hbm.at[pl.ds(base, PER_W)], idx_v)

    rows = (rows0, rows1)
    gsem = (gsem0, gsem1)
    ssem = (ssem0, ssem1)

    def fire_gathers(g, b):
        off = g * SUPER
        for k in range(K):
            pltpu.async_copy(
                table_hbm.at[idx_v.at[pl.ds(off + k * CHUNK, CHUNK)]],
                rows[b].at[pl.ds(k * CHUNK, CHUNK)],
                gsem[b],
            )

    def drain_gathers(b):
        pltpu.make_async_copy(
            table_hbm.at[pl.ds(0, SUPER)], rows[b], gsem[b]
        ).wait()

    def fire_store(g, b):
        pltpu.async_copy(
            rows[b],
            out_hbm.at[pl.ds(base + g * SUPER, SUPER), pl.ds(0, EMBED_DIM)],
            ssem[b],
        )

    def drain_store(b):
        pltpu.make_async_copy(
            rows[b],
            out_hbm.at[pl.ds(base, SUPER), pl.ds(0, EMBED_DIM)],
            ssem[b],
        ).wait()

    fire_gathers(0, 0)

    @pl.loop(0, G, step=2)
    def _(go):
        for b in (0, 1):
            g = go + b
            nb = 1 - b

            @pl.when(g + 1 < G)
            def _():
                @pl.when(g >= 1)
                def _():
                    drain_store(nb)
                fire_gathers(g + 1, nb)

            drain_gathers(b)
            fire_store(g, b)

    drain_store(0)
    drain_store(1)


_mesh = plsc.VectorSubcoreMesh(core_axis_name="c", subcore_axis_name="s")

_embed = pl.kernel(
    _embed_body,
    out_type=jax.ShapeDtypeStruct((B_TOTAL, PAD_DIM), jnp.float32),
    mesh=_mesh,
    scratch_types=[
        pltpu.VMEM((PER_W,), jnp.int32),
        pltpu.VMEM((SUPER, EMBED_DIM), jnp.float32),
        pltpu.VMEM((SUPER, EMBED_DIM), jnp.float32),
        pltpu.SemaphoreType.DMA,
        pltpu.SemaphoreType.DMA,
        pltpu.SemaphoreType.DMA,
        pltpu.SemaphoreType.DMA,
    ],
    compiler_params=pltpu.CompilerParams(use_tc_tiling_on_sc=False),
)


@jax.jit
def kernel(input_sentence, word_embedding):
    S, T = input_sentence.shape
    idx = input_sentence.reshape(-1).astype(jnp.int32)
    out = _embed(word_embedding, idx)
    return out[:, :EMBED_DIM].reshape(S, T, EMBED_DIM)
